# baseline (speedup 1.0000x reference)
import functools

import jax
import jax.numpy as jnp
from jax import lax
from jax.experimental import pallas as pl
from jax.experimental.pallas import tpu as pltpu
from jax.experimental.pallas import tpu_sc as plsc

F_IN = 480
F_TGT = 512
D = 32
L = 16


def _sc_body(ll_hbm, perm_hbm, out_hbm, perm_v, inv1d, inv2d, buf, sem, nb):
    cid = lax.axis_index("c")
    sid = lax.axis_index("s")
    wid = sid * 2 + cid

    pltpu.sync_copy(perm_hbm, perm_v.at[pl.ds(0, F_TGT)])
    lane0 = lax.iota(jnp.int32, L) == 0

    def istep(j, carry):
        p = perm_v[pl.ds(j, L)][0]
        w = inv1d[pl.ds(p, L)]
        inv1d[pl.ds(p, L)] = jnp.where(lane0, j, w)
        return carry

    lax.fori_loop(0, F_TGT, istep, 0)
    for c in range(4):
        for h in range(128 // L):
            inv2d[c, pl.ds(h * L, L)] = inv1d[pl.ds(c * 128 + h * L, L)]

    zero = jnp.zeros((L,), jnp.float32)
    for r in range(F_IN, F_TGT):
        buf[r, pl.ds(0, L)] = zero
        buf[r, pl.ds(L, L)] = zero

    base = wid * nb

    def bstep(i, carry):
        b = base + i
        pltpu.sync_copy(ll_hbm.at[b], buf.at[pl.ds(0, F_IN)])
        for c in range(4):
            pltpu.async_copy(
                buf.at[pl.ds(c * 128, 128)],
                out_hbm.at[b].at[inv2d.at[c]],
                sem,
            ).wait()
        return carry

    lax.fori_loop(0, nb, bstep, 0)


def kernel(ll, perm):
    b, f, c, r = ll.shape
    f_tgt = perm.shape[0]
    ll3 = ll.reshape(b, f, c * r)

    nw = 32
    nb = b // nw

    mesh = plsc.VectorSubcoreMesh(core_axis_name="c", subcore_axis_name="s")
    run = pl.kernel(
        functools.partial(_sc_body, nb=nb),
        out_type=jax.ShapeDtypeStruct((b, f_tgt, c * r), jnp.float32),
        mesh=mesh,
        compiler_params=pltpu.CompilerParams(use_tc_tiling_on_sc=False),
        scratch_types=[
            pltpu.VMEM((f_tgt + 16,), jnp.int32),
            pltpu.VMEM((f_tgt + 16,), jnp.int32),
            pltpu.VMEM((4, 128), jnp.int32),
            pltpu.VMEM((f_tgt, c * r), jnp.float32),
            pltpu.SemaphoreType.DMA,
        ],
    )
    out3 = run(ll3, perm)
    return out3.reshape(b, f_tgt, c, r)


# speedup vs baseline: 1.0423x; 1.0423x over previous
import functools

import jax
import jax.numpy as jnp
from jax import lax
from jax.experimental import pallas as pl
from jax.experimental.pallas import tpu as pltpu
from jax.experimental.pallas import tpu_sc as plsc

F_IN = 480
F_TGT = 512
D = 32
L = 16
NBB = 2


def _sc_body(ll_hbm, perm_hbm, out_hbm, perm_v, inv1d, inv2d, buf,
             gsem0, gsem1, ssem0, ssem1, nb):
    cid = lax.axis_index("c")
    sid = lax.axis_index("s")
    wid = sid * 2 + cid
    gsem = (gsem0, gsem1)
    ssem = (ssem0, ssem1)

    pltpu.sync_copy(perm_hbm, perm_v.at[pl.ds(0, F_TGT)])
    lane0 = lax.iota(jnp.int32, L) == 0

    def istep(j, carry):
        p = perm_v[pl.ds(j, L)][0]
        w = inv1d[pl.ds(p, L)]
        inv1d[pl.ds(p, L)] = jnp.where(lane0, j, w)
        return carry

    lax.fori_loop(0, F_TGT, istep, 0)
    for c in range(4):
        for h in range(128 // L):
            inv2d[c, pl.ds(h * L, L)] = inv1d[pl.ds(c * 128 + h * L, L)]

    zero = jnp.zeros((L,), jnp.float32)
    for s in range(2):
        for bb in range(NBB):
            for r in range(F_IN, F_TGT):
                buf[s, bb, r, pl.ds(0, L)] = zero
                buf[s, bb, r, pl.ds(L, L)] = zero

    base = wid * nb
    npairs = nb // (2 * NBB)

    def start_g(s, step):
        b = base + step * NBB
        pltpu.async_copy(
            ll_hbm.at[pl.ds(b, NBB)],
            buf.at[s].at[:, pl.ds(0, F_IN)],
            gsem[s],
        )

    def drain_g(s):
        pltpu.make_async_copy(
            ll_hbm.at[pl.ds(0, NBB)],
            buf.at[s].at[:, pl.ds(0, F_IN)],
            gsem[s],
        ).wait()

    def start_s(s, step):
        b = base + step * NBB
        for bb in range(NBB):
            for c in range(4):
                pltpu.async_copy(
                    buf.at[s, bb, pl.ds(c * 128, 128)],
                    out_hbm.at[b + bb].at[inv2d.at[c]],
                    ssem[s],
                )

    def drain_s(s):
        pltpu.make_async_copy(out_hbm.at[pl.ds(0, NBB)], buf.at[s], ssem[s]).wait()

    start_g(0, 0)

    def pair(t, carry):
        g = t * 2
        drain_g(0)
        start_s(0, g)

        @pl.when(t > 0)
        def _():
            drain_s(1)

        start_g(1, g + 1)
        drain_g(1)
        start_s(1, g + 1)
        drain_s(0)

        @pl.when(t + 1 < npairs)
        def _():
            start_g(0, g + 2)

        return carry

    lax.fori_loop(0, npairs, pair, 0)
    drain_s(1)


def kernel(ll, perm):
    b, f, c, r = ll.shape
    f_tgt = perm.shape[0]
    ll3 = ll.reshape(b, f, c * r)

    nw = 32
    nb = b // nw

    mesh = plsc.VectorSubcoreMesh(core_axis_name="c", subcore_axis_name="s")
    run = pl.kernel(
        functools.partial(_sc_body, nb=nb),
        out_type=jax.ShapeDtypeStruct((b, f_tgt, c * r), jnp.float32),
        mesh=mesh,
        compiler_params=pltpu.CompilerParams(use_tc_tiling_on_sc=False),
        scratch_types=[
            pltpu.VMEM((f_tgt + 16,), jnp.int32),
            pltpu.VMEM((f_tgt + 16,), jnp.int32),
            pltpu.VMEM((4, 128), jnp.int32),
            pltpu.VMEM((2, NBB, f_tgt, c * r), jnp.float32),
            pltpu.SemaphoreType.DMA,
            pltpu.SemaphoreType.DMA,
            pltpu.SemaphoreType.DMA,
            pltpu.SemaphoreType.DMA,
        ],
    )
    out3 = run(ll3, perm)
    return out3.reshape(b, f_tgt, c, r)


# speedup vs baseline: 2.6981x; 2.5886x over previous
import jax
import jax.numpy as jnp
from jax.experimental import pallas as pl
from jax.experimental.pallas import tpu as pltpu

F_IN = 480
F_TGT = 512
B = 4096
C = 8
R = 4
NBT = B // 128
NFT = F_TGT // 128
SLAB = NBT * R * 128
NSTEPS = C * NFT


def _tc_body(perm_ref, x_ref, y_ref, buf, yblk, gsem0, gsem1, wsem0, wsem1):
    gsem = (gsem0, gsem1)
    wsem = (wsem0, wsem1)
    p = pl.program_id(0)
    npairs = pl.num_programs(0)

    def start_gather(sl, step):
        cc = step // NFT
        ftc = step % NFT
        for fl in range(128):
            pv = perm_ref[ftc * 128 + fl]

            @pl.when(pv < F_IN)
            def _():
                pltpu.make_async_copy(
                    x_ref.at[pv, cc], buf.at[sl, fl], gsem[sl]
                ).start()

    def finish_gather(sl, step):
        cc = step // NFT
        ftc = step % NFT
        for fl in range(128):
            pv = perm_ref[ftc * 128 + fl]

            @pl.when(pv < F_IN)
            def _():
                pltpu.make_async_copy(
                    x_ref.at[pv, cc], buf.at[sl, fl], gsem[sl]
                ).wait()

            @pl.when(pv >= F_IN)
            def _():
                buf[sl, fl, :] = jnp.zeros((SLAB,), jnp.float32)

    def compute_and_write(sl, step):
        for bt in range(NBT):
            for r in range(R):
                t = buf[sl, :, pl.ds((bt * R + r) * 128, 128)]
                yblk[sl, pl.ds(bt * 128, 128), pl.ds(r * 128, 128)] = t.T
        pltpu.make_async_copy(yblk.at[sl], y_ref.at[:, step, :], wsem[sl]).start()

    def drain_write(sl):
        pltpu.make_async_copy(yblk.at[sl], y_ref.at[:, 0, :], wsem[sl]).wait()

    a = p * 2

    @pl.when(p == 0)
    def _():
        start_gather(0, a)

    start_gather(1, a + 1)

    @pl.when(p >= 1)
    def _():
        drain_write(0)

    finish_gather(0, a)
    compute_and_write(0, a)

    @pl.when(p + 1 < npairs)
    def _():
        start_gather(0, a + 2)

    @pl.when(p >= 1)
    def _():
        drain_write(1)

    finish_gather(1, a + 1)
    compute_and_write(1, a + 1)

    @pl.when(p == npairs - 1)
    def _():
        drain_write(0)
        drain_write(1)


def kernel(ll, perm):
    b, f, c, r = ll.shape
    f_tgt = perm.shape[0]

    x3 = (
        ll.reshape(NBT, 128, f, c, r)
        .transpose(2, 3, 0, 4, 1)
        .reshape(f, c, SLAB)
    )

    y3 = pl.pallas_call(
        _tc_body,
        grid=(NSTEPS // 2,),
        in_specs=[
            pl.BlockSpec(memory_space=pltpu.SMEM),
            pl.BlockSpec(memory_space=pl.ANY),
        ],
        out_specs=pl.BlockSpec(memory_space=pl.ANY),
        out_shape=jax.ShapeDtypeStruct((b, NSTEPS, 512), jnp.float32),
        scratch_shapes=[
            pltpu.VMEM((2, 128, SLAB), jnp.float32),
            pltpu.VMEM((2, B, 512), jnp.float32),
            pltpu.SemaphoreType.DMA,
            pltpu.SemaphoreType.DMA,
            pltpu.SemaphoreType.DMA,
            pltpu.SemaphoreType.DMA,
        ],
        compiler_params=pltpu.CompilerParams(
            dimension_semantics=("arbitrary",),
            vmem_limit_bytes=110 * 1024 * 1024,
        ),
    )(perm, x3)

    return (
        y3.reshape(b, c, NFT, r, 128)
        .transpose(0, 2, 4, 1, 3)
        .reshape(b, f_tgt, c, r)
    )


# speedup vs baseline: 5.2420x; 1.9429x over previous
import jax
import jax.numpy as jnp
from jax.experimental import pallas as pl
from jax.experimental.pallas import tpu as pltpu

F_IN = 480
F_TGT = 512
B = 4096
C = 8
R = 4
NBT = B // 128
NFT = F_TGT // 128
SLAB = NBT * R * 128
NSTEPS = C * NFT


def _tc_body(perm_ref, x_ref, y_ref, buf, yblk, gsem0, gsem1, wsem0, wsem1):
    gsem = (gsem0, gsem1)
    wsem = (wsem0, wsem1)
    p = pl.program_id(0)
    npairs = pl.num_programs(0)

    def start_gather(sl, step):
        cc = step // NFT
        ftc = step % NFT
        for fl in range(128):
            pv = perm_ref[ftc * 128 + fl]

            @pl.when(pv < F_IN)
            def _():
                pltpu.make_async_copy(
                    x_ref.at[pl.ds((pv * C + cc) * 128, 128)],
                    buf.at[sl, fl],
                    gsem[sl],
                ).start()

    def finish_gather(sl, step):
        cc = step // NFT
        ftc = step % NFT
        for fl in range(128):
            pv = perm_ref[ftc * 128 + fl]

            @pl.when(pv < F_IN)
            def _():
                pltpu.make_async_copy(
                    x_ref.at[pl.ds((pv * C + cc) * 128, 128)],
                    buf.at[sl, fl],
                    gsem[sl],
                ).wait()

            @pl.when(pv >= F_IN)
            def _():
                buf[sl, fl, :, :] = jnp.zeros((NBT * R, 128), jnp.float32)

    def compute_and_write(sl, step):
        ws = sl % 2
        for bt in range(NBT):
            for r in range(R):
                t = buf[sl, :, bt * R + r, :]
                yblk[ws, pl.ds(bt * 128, 128), pl.ds(r * 128, 128)] = t.T
        pltpu.make_async_copy(yblk.at[ws], y_ref.at[:, step, :], wsem[ws]).start()

    def drain_write(sl):
        pltpu.make_async_copy(yblk.at[sl], y_ref.at[:, 0, :], wsem[sl]).wait()

    a = p * 2

    @pl.when(p == 0)
    def _():
        start_gather(0, a)

    start_gather(1, a + 1)

    @pl.when(p >= 1)
    def _():
        drain_write(0)

    finish_gather(0, a)
    compute_and_write(0, a)

    @pl.when(p + 1 < npairs)
    def _():
        start_gather(0, a + 2)

    @pl.when(p >= 1)
    def _():
        drain_write(1)

    finish_gather(1, a + 1)
    compute_and_write(1, a + 1)

    @pl.when(p == npairs - 1)
    def _():
        drain_write(0)
        drain_write(1)


def kernel(ll, perm):
    b, f, c, r = ll.shape
    f_tgt = perm.shape[0]

    x2 = (
        ll.reshape(NBT, 128, f, c, r)
        .transpose(2, 3, 0, 4, 1)
        .reshape(f * c * NBT * r, 128)
    )

    y3 = pl.pallas_call(
        _tc_body,
        grid=(NSTEPS // 2,),
        in_specs=[
            pl.BlockSpec(memory_space=pltpu.SMEM),
            pl.BlockSpec(memory_space=pl.ANY),
        ],
        out_specs=pl.BlockSpec(memory_space=pl.ANY),
        out_shape=jax.ShapeDtypeStruct((b, NSTEPS, 512), jnp.float32),
        scratch_shapes=[
            pltpu.VMEM((2, 128, NBT * R, 128), jnp.float32),
            pltpu.VMEM((2, B, 512), jnp.float32),
            pltpu.SemaphoreType.DMA,
            pltpu.SemaphoreType.DMA,
            pltpu.SemaphoreType.DMA,
            pltpu.SemaphoreType.DMA,
        ],
        compiler_params=pltpu.CompilerParams(
            dimension_semantics=("arbitrary",),
            vmem_limit_bytes=110 * 1024 * 1024,
        ),
    )(perm, x2)

    return (
        y3.reshape(b, c, NFT, r, 128)
        .transpose(0, 2, 4, 1, 3)
        .reshape(b, f_tgt, c, r)
    )
